# baseline (speedup 1.0000x reference)
import jax
import jax.numpy as jnp
from jax import lax
from jax.experimental import pallas as pl
from jax.experimental.pallas import tpu as pltpu
from jax.experimental.pallas import tpu_sc as plsc

_NCAMS = 100
_NIDENT = 5000
_B = 32
_ROWS = _B * 3
_LANES = 16
_CHUNKS = _ROWS // _LANES


def _sc_params_body(cam_hbm, id_hbm, wcamf, bcamf, widentf, bidentf, wf, bf,
                    wv_hbm, bv_hbm,
                    cam_v, id_v, icam_v, iid_v, iwb_v,
                    g_wcam, g_wident, g_w, g_bcam, g_bident, g_b,
                    wv_v, bv_v, sem):
    is_lead = (lax.axis_index("c") == 0) & (lax.axis_index("s") == 0)

    @pl.when(is_lead)
    def _():
        pltpu.sync_copy(cam_hbm, cam_v)
        pltpu.sync_copy(id_hbm, id_v)
        for k in range(_CHUNKS):
            c = k // 2
            boff = _LANES * (k % 2)
            sl = pl.ds(_LANES * k, _LANES)
            camb = cam_v[pl.ds(boff, _LANES)]
            idb = id_v[pl.ds(boff, _LANES)]
            icam_v[sl] = camb * 3 + c
            iid_v[sl] = idb * 3 + c
            iwb_v[sl] = (camb * _NIDENT + idb) * 3 + c
        cps = [
            pltpu.async_copy(wcamf.at[icam_v], g_wcam, sem),
            pltpu.async_copy(widentf.at[iid_v], g_wident, sem),
            pltpu.async_copy(wf.at[iwb_v], g_w, sem),
            pltpu.async_copy(bcamf.at[icam_v], g_bcam, sem),
            pltpu.async_copy(bidentf.at[iid_v], g_bident, sem),
            pltpu.async_copy(bf.at[iwb_v], g_b, sem),
        ]
        for cp in cps:
            cp.wait()
        for k in range(_CHUNKS):
            sl = pl.ds(_LANES * k, _LANES)
            wv_v[sl] = g_wcam[sl] + g_wident[sl] + 10.0 * g_w[sl]
            bv_v[sl] = g_bcam[sl] + g_bident[sl] + 10.0 * g_b[sl]
        pltpu.sync_copy(wv_v, wv_hbm)
        pltpu.sync_copy(bv_v, bv_hbm)


_sc_params = pl.kernel(
    _sc_params_body,
    out_type=[jax.ShapeDtypeStruct((_ROWS,), jnp.float32),
              jax.ShapeDtypeStruct((_ROWS,), jnp.float32)],
    mesh=plsc.VectorSubcoreMesh(core_axis_name="c", subcore_axis_name="s"),
    scratch_types=[
        pltpu.VMEM((_B,), jnp.int32),
        pltpu.VMEM((_B,), jnp.int32),
        pltpu.VMEM((_ROWS,), jnp.int32),
        pltpu.VMEM((_ROWS,), jnp.int32),
        pltpu.VMEM((_ROWS,), jnp.int32),
        pltpu.VMEM((_ROWS,), jnp.float32),
        pltpu.VMEM((_ROWS,), jnp.float32),
        pltpu.VMEM((_ROWS,), jnp.float32),
        pltpu.VMEM((_ROWS,), jnp.float32),
        pltpu.VMEM((_ROWS,), jnp.float32),
        pltpu.VMEM((_ROWS,), jnp.float32),
        pltpu.VMEM((_ROWS,), jnp.float32),
        pltpu.VMEM((_ROWS,), jnp.float32),
        pltpu.SemaphoreType.DMA,
    ],
)

_G = 4


def _scale_body(wv_ref, bv_ref, img_ref, out_ref):
    i = pl.program_id(0)
    for j in range(_G):
        r = i * _G + j
        b = r // 3
        c = r - 3 * b
        q = c * _B + b
        out_ref[j] = img_ref[j] * wv_ref[q] + bv_ref[q]


def _scale(wv, bv, img, h, ww):
    return pl.pallas_call(
        _scale_body,
        grid=(_ROWS // _G,),
        in_specs=[
            pl.BlockSpec(memory_space=pltpu.SMEM),
            pl.BlockSpec(memory_space=pltpu.SMEM),
            pl.BlockSpec((_G, h, ww), lambda i: (i, 0, 0)),
        ],
        out_specs=pl.BlockSpec((_G, h, ww), lambda i: (i, 0, 0)),
        out_shape=jax.ShapeDtypeStruct((_ROWS, h, ww), jnp.float32),
    )(wv, bv, img)


def kernel(image, camindex, idindex, wcam, bcam, wident, bident, w, b):
    bsz, ch, h, ww = image.shape
    cam = camindex.astype(jnp.int32)
    idn = idindex.astype(jnp.int32)
    wv, bv = _sc_params(cam, idn,
                        wcam.reshape(-1), bcam.reshape(-1),
                        wident.reshape(-1), bident.reshape(-1),
                        w.reshape(-1), b.reshape(-1))
    out = _scale(wv, bv, image.reshape(bsz * ch, h, ww), h, ww)
    return out.reshape(bsz, ch, h, ww)


# speedup vs baseline: 9.7072x; 9.7072x over previous
import jax
import jax.numpy as jnp
from jax.experimental import pallas as pl
from jax.experimental.pallas import tpu as pltpu

_B = 32


def _body(cam_s, id_s, wcam_ref, bcam_ref, wident_ref, bident_ref,
          w_ref, b_ref, img_ref, out_ref):
    i = pl.program_id(0)
    rc = cam_s[i] % 8
    ri = id_s[i] % 8
    for c in range(3):
        wv = wcam_ref[rc, c] + wident_ref[ri, c] + 10.0 * w_ref[0, ri, c]
        bv = bcam_ref[rc, c] + bident_ref[ri, c] + 10.0 * b_ref[0, ri, c]
        out_ref[0, c] = img_ref[0, c] * wv + bv


def kernel(image, camindex, idindex, wcam, bcam, wident, bident, w, b):
    bsz, ch, h, ww = image.shape
    cam = camindex.astype(jnp.int32)
    idn = idindex.astype(jnp.int32)
    grid_spec = pltpu.PrefetchScalarGridSpec(
        num_scalar_prefetch=2,
        grid=(bsz,),
        in_specs=[
            pl.BlockSpec((8, 3), lambda i, cs, ids: (cs[i] // 8, 0)),
            pl.BlockSpec((8, 3), lambda i, cs, ids: (cs[i] // 8, 0)),
            pl.BlockSpec((8, 3), lambda i, cs, ids: (ids[i] // 8, 0)),
            pl.BlockSpec((8, 3), lambda i, cs, ids: (ids[i] // 8, 0)),
            pl.BlockSpec((1, 8, 3), lambda i, cs, ids: (cs[i], ids[i] // 8, 0)),
            pl.BlockSpec((1, 8, 3), lambda i, cs, ids: (cs[i], ids[i] // 8, 0)),
            pl.BlockSpec((1, ch, h, ww), lambda i, cs, ids: (i, 0, 0, 0)),
        ],
        out_specs=pl.BlockSpec((1, ch, h, ww), lambda i, cs, ids: (i, 0, 0, 0)),
    )
    return pl.pallas_call(
        _body,
        grid_spec=grid_spec,
        out_shape=jax.ShapeDtypeStruct(image.shape, jnp.float32),
    )(cam, idn, wcam, bcam, wident, bident, w, b, image)
